# baseline (speedup 1.0000x reference)
import jax
import jax.numpy as jnp
from jax import lax
from jax.experimental import pallas as pl
from jax.experimental.pallas import tpu as pltpu
from jax.experimental.pallas import tpu_sc as plsc

BATCH = 4096
SEQ = 50
EMB = 32
LANES = 16
NC, NS = 2, 16
NW = NC * NS
BPW = BATCH // NW
RPC = 2
CHUNKS = BPW // RPC
IPC = RPC * SEQ
IPAD = 104
TRASH = BPW
ACC_ROWS = BPW + 8
NBUF = 4


def _body(idx_hbm, seg_hbm, table_hbm, out_hbm,
          idx_v, seg_v, rows_v, zero_v, acc_sh, gsems, ssems):
    cid = lax.axis_index("c")
    sid = lax.axis_index("s")
    wid = sid * NC + cid

    pltpu.sync_copy(idx_hbm.at[wid], idx_v)
    pltpu.sync_copy(seg_hbm, seg_v)

    def gather(c, slot):
        pltpu.async_copy(table_hbm.at[idx_v.at[c]], rows_v.at[slot], gsems.at[slot])

    for b in range(NBUF):
        gather(b, b)

    zero = jnp.zeros((LANES,), jnp.float32)
    for r in range(ACC_ROWS):
        zero_v[r, pl.ds(0, LANES)] = zero
        zero_v[r, pl.ds(LANES, LANES)] = zero
    pltpu.sync_copy(zero_v, acc_sh.at[sid])

    def step(i, carry):
        for b in range(NBUF):
            c = i * NBUF + b
            pltpu.make_async_copy(
                table_hbm.at[idx_v.at[c]], rows_v.at[b], gsems.at[b]
            ).wait()
            pltpu.async_copy(
                rows_v.at[b], acc_sh.at[sid].at[seg_v.at[c]], ssems.at[b],
                add=True,
            )
            pltpu.make_async_copy(
                rows_v.at[b], acc_sh.at[sid].at[seg_v.at[c]], ssems.at[b]
            ).wait()
            nxt = c + NBUF

            @pl.when(nxt < CHUNKS)
            def _():
                gather(nxt, b)

        return carry

    lax.fori_loop(0, CHUNKS // NBUF, step, 0)

    pltpu.sync_copy(acc_sh.at[sid], zero_v)
    inv = jnp.full((LANES,), 1.0 / SEQ, jnp.float32)
    for r in range(BPW):
        zero_v[r, pl.ds(0, LANES)] = zero_v[r, pl.ds(0, LANES)] * inv
        zero_v[r, pl.ds(LANES, LANES)] = zero_v[r, pl.ds(LANES, LANES)] * inv
    pltpu.sync_copy(zero_v.at[pl.ds(0, BPW)], out_hbm.at[pl.ds(wid * BPW, BPW)])


def kernel(inputs, table):
    idx = inputs.astype(jnp.int32).reshape(NW, CHUNKS, IPC)
    idx = jnp.pad(idx, ((0, 0), (0, 0), (0, IPAD - IPC)))
    j = jnp.arange(IPAD, dtype=jnp.int32)
    base = jnp.arange(CHUNKS, dtype=jnp.int32)[:, None] * RPC
    seg = jnp.where(j[None, :] < IPC, base + j[None, :] // SEQ, TRASH)

    mesh = plsc.VectorSubcoreMesh(core_axis_name="c", subcore_axis_name="s")
    run = pl.kernel(
        _body,
        out_type=jax.ShapeDtypeStruct((BATCH, EMB), jnp.float32),
        mesh=mesh,
        scratch_types=[
            pltpu.VMEM((CHUNKS, IPAD), jnp.int32),
            pltpu.VMEM((CHUNKS, IPAD), jnp.int32),
            pltpu.VMEM((NBUF, IPAD, EMB), jnp.float32),
            pltpu.VMEM((ACC_ROWS, EMB), jnp.float32),
            pltpu.VMEM_SHARED((NS, ACC_ROWS, EMB), jnp.float32),
            pltpu.SemaphoreType.DMA((NBUF,)),
            pltpu.SemaphoreType.DMA((NBUF,)),
        ],
        compiler_params=pltpu.CompilerParams(use_tc_tiling_on_sc=False),
    )
    return run(idx, seg, table)


# speedup vs baseline: 1.0010x; 1.0010x over previous
import jax
import jax.numpy as jnp
from jax import lax
from jax.experimental import pallas as pl
from jax.experimental.pallas import tpu as pltpu
from jax.experimental.pallas import tpu_sc as plsc

BATCH = 4096
SEQ = 50
EMB = 32
LANES = 16
NC, NS = 2, 16
NW = NC * NS
BPW = BATCH // NW
RPC = 2
CHUNKS = BPW // RPC
IPC = RPC * SEQ
IPAD = 104
TRASH = BPW
ACC_ROWS = BPW + 8
NBUF = 8


def _body(idx_hbm, seg_hbm, table_hbm, out_hbm,
          idx_v, seg_v, rows_v, zero_v, acc_sh, gsems, ssems):
    cid = lax.axis_index("c")
    sid = lax.axis_index("s")
    wid = sid * NC + cid

    pltpu.sync_copy(idx_hbm.at[wid], idx_v)
    pltpu.sync_copy(seg_hbm, seg_v)

    def gather(c, slot):
        pltpu.async_copy(table_hbm.at[idx_v.at[c]], rows_v.at[slot], gsems.at[slot])

    for b in range(NBUF):
        gather(b, b)

    zero = jnp.zeros((LANES,), jnp.float32)
    for r in range(ACC_ROWS):
        zero_v[r, pl.ds(0, LANES)] = zero
        zero_v[r, pl.ds(LANES, LANES)] = zero
    pltpu.sync_copy(zero_v, acc_sh.at[sid])

    def step(i, carry):
        for b in range(NBUF):
            c = i * NBUF + b
            pltpu.make_async_copy(
                table_hbm.at[idx_v.at[c]], rows_v.at[b], gsems.at[b]
            ).wait()
            pltpu.async_copy(
                rows_v.at[b], acc_sh.at[sid].at[seg_v.at[c]], ssems.at[b],
                add=True,
            )
            pltpu.make_async_copy(
                rows_v.at[b], acc_sh.at[sid].at[seg_v.at[c]], ssems.at[b]
            ).wait()
            nxt = c + NBUF

            @pl.when(nxt < CHUNKS)
            def _():
                gather(nxt, b)

        return carry

    lax.fori_loop(0, CHUNKS // NBUF, step, 0)

    pltpu.sync_copy(acc_sh.at[sid], zero_v)
    inv = jnp.full((LANES,), 1.0 / SEQ, jnp.float32)
    for r in range(BPW):
        zero_v[r, pl.ds(0, LANES)] = zero_v[r, pl.ds(0, LANES)] * inv
        zero_v[r, pl.ds(LANES, LANES)] = zero_v[r, pl.ds(LANES, LANES)] * inv
    pltpu.sync_copy(zero_v.at[pl.ds(0, BPW)], out_hbm.at[pl.ds(wid * BPW, BPW)])


def kernel(inputs, table):
    idx = inputs.astype(jnp.int32).reshape(NW, CHUNKS, IPC)
    idx = jnp.pad(idx, ((0, 0), (0, 0), (0, IPAD - IPC)))
    j = jnp.arange(IPAD, dtype=jnp.int32)
    base = jnp.arange(CHUNKS, dtype=jnp.int32)[:, None] * RPC
    seg = jnp.where(j[None, :] < IPC, base + j[None, :] // SEQ, TRASH)

    mesh = plsc.VectorSubcoreMesh(core_axis_name="c", subcore_axis_name="s")
    run = pl.kernel(
        _body,
        out_type=jax.ShapeDtypeStruct((BATCH, EMB), jnp.float32),
        mesh=mesh,
        scratch_types=[
            pltpu.VMEM((CHUNKS, IPAD), jnp.int32),
            pltpu.VMEM((CHUNKS, IPAD), jnp.int32),
            pltpu.VMEM((NBUF, IPAD, EMB), jnp.float32),
            pltpu.VMEM((ACC_ROWS, EMB), jnp.float32),
            pltpu.VMEM_SHARED((NS, ACC_ROWS, EMB), jnp.float32),
            pltpu.SemaphoreType.DMA((NBUF,)),
            pltpu.SemaphoreType.DMA((NBUF,)),
        ],
        compiler_params=pltpu.CompilerParams(use_tc_tiling_on_sc=False),
    )
    return run(idx, seg, table)
